# baseline (speedup 1.0000x reference)
import functools

import jax
import jax.numpy as jnp
from jax import lax
from jax.experimental import pallas as pl
from jax.experimental.pallas import tpu as pltpu
from jax.experimental.pallas import tpu_sc as plsc

NUM_ITEMS = 8192
DIM = 10000
NLANE = DIM // 16
EPS = 1e-8

NTILE = 32
RPT = NUM_ITEMS // NTILE
SC_CHUNK = 4
NCHUNK = RPT // SC_CHUNK

_sc_mesh = plsc.VectorSubcoreMesh(core_axis_name="c", subcore_axis_name="s")


@functools.partial(
    pl.kernel,
    out_type=[
        jax.ShapeDtypeStruct((NTILE, DIM), jnp.float32),
        jax.ShapeDtypeStruct((NTILE, 48), jnp.float32),
    ],
    mesh=_sc_mesh,
    compiler_params=pltpu.CompilerParams(needs_layout_passes=False),
    scratch_types=[
        pltpu.VMEM((DIM,), jnp.float32),
        pltpu.VMEM((SC_CHUNK * DIM,), jnp.float32),
        pltpu.VMEM((SC_CHUNK * DIM,), jnp.float32),
        pltpu.VMEM((DIM,), jnp.float32),
        pltpu.VMEM((48,), jnp.float32),
        pltpu.SMEM((1,), jnp.float32),
        pltpu.SMEM((1,), jnp.int32),
        pltpu.SemaphoreType.DMA,
        pltpu.SemaphoreType.DMA,
    ],
)
def _sc_scan(noisy_hbm, vec_hbm, rows_out, meta_out,
             noisy_v, bufa, bufb, cand_v, meta_v,
             best_s, bidx_s, sema, semb):
    cid = lax.axis_index("c")
    sid = lax.axis_index("s")
    wid = sid * 2 + cid
    base = wid * RPT
    pltpu.sync_copy(noisy_hbm, noisy_v)
    best_s[0] = -jnp.inf
    bidx_s[0] = 0
    bufs = (bufa, bufb)
    sems = (sema, semb)
    cwords = SC_CHUNK * DIM

    def start(ci, s):
        pltpu.make_async_copy(
            vec_hbm.at[pl.ds((base + ci * SC_CHUNK) * DIM, cwords)],
            bufs[s], sems[s]).start()

    start(0, 0)
    start(1, 1)

    def outer(o, _):
        for s in range(2):
            ci = o * 2 + s
            pltpu.make_async_copy(
                vec_hbm.at[pl.ds((base + ci * SC_CHUNK) * DIM, cwords)],
                bufs[s], sems[s]).wait()
            buf = bufs[s]

            def jbody(j, accs, buf=buf):
                nj = noisy_v[pl.ds(j * 16, 16)]
                return tuple(a + buf[pl.ds(r * DIM + j * 16, 16)] * nj
                             for r, a in enumerate(accs))

            accs = lax.fori_loop(
                0, NLANE, jbody,
                tuple(jnp.zeros((16,), jnp.float32)
                      for _ in range(SC_CHUNK)))
            for r in range(SC_CHUNK):
                dot_r = jnp.sum(accs[r])

                @pl.when(dot_r > best_s[0])
                def _update(dot_r=dot_r, ci=ci, r=r, buf=buf):
                    best_s[0] = dot_r
                    bidx_s[0] = base + ci * SC_CHUNK + r

                    def cbody(j, sq, buf=buf, r=r):
                        v = buf[pl.ds(r * DIM + j * 16, 16)]
                        cand_v[pl.ds(j * 16, 16)] = v
                        return sq + v * v

                    sq = lax.fori_loop(0, NLANE, cbody,
                                       jnp.zeros((16,), jnp.float32))
                    meta_v[pl.ds(16, 16)] = sq

            @pl.when(ci + 2 < NCHUNK)
            def _refill(ci=ci, s=s):
                start(ci + 2, s)
        return 0

    lax.fori_loop(0, NCHUNK // 2, outer, 0)

    meta_v[pl.ds(0, 16)] = jnp.broadcast_to(best_s[0], (16,))
    meta_v[pl.ds(32, 16)] = jnp.broadcast_to(
        bidx_s[0].astype(jnp.float32), (16,))
    pltpu.sync_copy(cand_v, rows_out.at[wid])
    pltpu.sync_copy(meta_v, meta_out.at[wid])


def _merge_body(meta_ref, rows_ref, noisy_ref, clean_ref, idx_ref, sim_ref):
    a = meta_ref[...]
    dotc = a[:, 0:1]
    idxc = a[:, 32:33]
    m = jnp.max(dotc)
    big = jnp.float32(NUM_ITEMS)
    bi_f = jnp.min(jnp.where(dotc == m, idxc, big))
    tilei = lax.broadcasted_iota(jnp.int32, (NTILE, 1), 0)
    wt = jnp.min(jnp.where((dotc == m) & (idxc == bi_f), tilei, NTILE))
    ssq = jnp.sum(a[:, 16:32] * (tilei == wt).astype(jnp.float32))
    c = rows_ref[pl.ds(wt, 1), :]
    clean_ref[...] = c
    n = noisy_ref[...]
    cnorm = jnp.maximum(jnp.sqrt(ssq), EPS)
    nn = jnp.maximum(jnp.sqrt(jnp.sum(n * n)), EPS)
    idx_ref[0, 0] = bi_f.astype(jnp.int32)
    sim_ref[0, 0] = m / (cnorm * nn)


@jax.jit
def kernel(noisy, vectors):
    rows, meta = _sc_scan(noisy, vectors.reshape(-1))
    clean, idx, sim = pl.pallas_call(
        _merge_body,
        in_specs=[
            pl.BlockSpec((NTILE, 48), lambda: (0, 0)),
            pl.BlockSpec((NTILE, DIM), lambda: (0, 0)),
            pl.BlockSpec((1, DIM), lambda: (0, 0)),
        ],
        out_specs=[
            pl.BlockSpec((1, DIM), lambda: (0, 0)),
            pl.BlockSpec(memory_space=pltpu.SMEM),
            pl.BlockSpec(memory_space=pltpu.SMEM),
        ],
        out_shape=[
            jax.ShapeDtypeStruct((1, DIM), jnp.float32),
            jax.ShapeDtypeStruct((1, 1), jnp.int32),
            jax.ShapeDtypeStruct((1, 1), jnp.float32),
        ],
    )(meta, rows, noisy.reshape(1, DIM))
    return clean[0], idx[0, 0], sim[0, 0]


# speedup vs baseline: 1.4934x; 1.4934x over previous
import functools

import jax
import jax.numpy as jnp
from jax import lax
from jax.experimental import pallas as pl
from jax.experimental.pallas import tpu as pltpu
from jax.experimental.pallas import tpu_sc as plsc

NUM_ITEMS = 8192
DIM = 10000
NLANE = DIM // 16
EPS = 1e-8

NTILE = 32
RPT = NUM_ITEMS // NTILE
SC_CHUNK = 4
NCHUNK = RPT // SC_CHUNK

_sc_mesh = plsc.VectorSubcoreMesh(core_axis_name="c", subcore_axis_name="s")


@functools.partial(
    pl.kernel,
    out_type=[
        jax.ShapeDtypeStruct((NTILE, DIM), jnp.float32),
        jax.ShapeDtypeStruct((NTILE, 48), jnp.float32),
    ],
    mesh=_sc_mesh,
    compiler_params=pltpu.CompilerParams(needs_layout_passes=False),
    scratch_types=[
        pltpu.VMEM((DIM,), jnp.float32),
        pltpu.VMEM((SC_CHUNK, DIM), jnp.float32),
        pltpu.VMEM((SC_CHUNK, DIM), jnp.float32),
        pltpu.VMEM((DIM,), jnp.float32),
        pltpu.VMEM((48,), jnp.float32),
        pltpu.SMEM((1,), jnp.float32),
        pltpu.SMEM((1,), jnp.int32),
        pltpu.SemaphoreType.DMA,
        pltpu.SemaphoreType.DMA,
    ],
)
def _sc_scan(noisy_hbm, vec_hbm, rows_out, meta_out,
             noisy_v, bufa, bufb, cand_v, meta_v,
             best_s, bidx_s, sema, semb):
    cid = lax.axis_index("c")
    sid = lax.axis_index("s")
    wid = sid * 2 + cid
    base = wid * RPT
    pltpu.sync_copy(noisy_hbm, noisy_v)
    best_s[0] = -jnp.inf
    bidx_s[0] = 0
    bufs = (bufa, bufb)
    sems = (sema, semb)

    def start(ci, s):
        pltpu.make_async_copy(
            vec_hbm.at[pl.ds(base + ci * SC_CHUNK, SC_CHUNK), :],
            bufs[s], sems[s]).start()

    start(0, 0)
    start(1, 1)

    def outer(o, _):
        for s in range(2):
            ci = o * 2 + s
            pltpu.make_async_copy(
                vec_hbm.at[pl.ds(base + ci * SC_CHUNK, SC_CHUNK), :],
                bufs[s], sems[s]).wait()
            buf = bufs[s]

            def jbody(j, accs, buf=buf):
                nj = noisy_v[pl.ds(j * 16, 16)]
                return tuple(a + buf[r, pl.ds(j * 16, 16)] * nj
                             for r, a in enumerate(accs))

            accs = lax.fori_loop(
                0, NLANE, jbody,
                tuple(jnp.zeros((16,), jnp.float32)
                      for _ in range(SC_CHUNK)))
            for r in range(SC_CHUNK):
                dot_r = jnp.sum(accs[r])

                @pl.when(dot_r > best_s[0])
                def _update(dot_r=dot_r, ci=ci, r=r, buf=buf):
                    best_s[0] = dot_r
                    bidx_s[0] = base + ci * SC_CHUNK + r

                    def cbody(j, sq, buf=buf, r=r):
                        v = buf[r, pl.ds(j * 16, 16)]
                        cand_v[pl.ds(j * 16, 16)] = v
                        return sq + v * v

                    sq = lax.fori_loop(0, NLANE, cbody,
                                       jnp.zeros((16,), jnp.float32))
                    meta_v[pl.ds(16, 16)] = sq

            @pl.when(ci + 2 < NCHUNK)
            def _refill(ci=ci, s=s):
                start(ci + 2, s)
        return 0

    lax.fori_loop(0, NCHUNK // 2, outer, 0)

    meta_v[pl.ds(0, 16)] = jnp.broadcast_to(best_s[0], (16,))
    meta_v[pl.ds(32, 16)] = jnp.broadcast_to(
        bidx_s[0].astype(jnp.float32), (16,))
    pltpu.sync_copy(cand_v, rows_out.at[wid])
    pltpu.sync_copy(meta_v, meta_out.at[wid])


def _merge_body(meta_ref, rows_ref, noisy_ref, clean_ref, idx_ref, sim_ref):
    a = meta_ref[...]
    dotc = a[:, 0:1]
    idxc = a[:, 32:33]
    m = jnp.max(dotc)
    big = jnp.float32(NUM_ITEMS)
    bi_f = jnp.min(jnp.where(dotc == m, idxc, big))
    tilei = lax.broadcasted_iota(jnp.int32, (NTILE, 1), 0)
    wt = jnp.min(jnp.where((dotc == m) & (idxc == bi_f), tilei, NTILE))
    ssq = jnp.sum(a[:, 16:32] * (tilei == wt).astype(jnp.float32))
    c = rows_ref[pl.ds(wt, 1), :]
    clean_ref[...] = c
    n = noisy_ref[...]
    cnorm = jnp.maximum(jnp.sqrt(ssq), EPS)
    nn = jnp.maximum(jnp.sqrt(jnp.sum(n * n)), EPS)
    idx_ref[0, 0] = bi_f.astype(jnp.int32)
    sim_ref[0, 0] = m / (cnorm * nn)


@jax.jit
def kernel(noisy, vectors):
    rows, meta = _sc_scan(noisy, vectors)
    clean, idx, sim = pl.pallas_call(
        _merge_body,
        in_specs=[
            pl.BlockSpec((NTILE, 48), lambda: (0, 0)),
            pl.BlockSpec((NTILE, DIM), lambda: (0, 0)),
            pl.BlockSpec((1, DIM), lambda: (0, 0)),
        ],
        out_specs=[
            pl.BlockSpec((1, DIM), lambda: (0, 0)),
            pl.BlockSpec(memory_space=pltpu.SMEM),
            pl.BlockSpec(memory_space=pltpu.SMEM),
        ],
        out_shape=[
            jax.ShapeDtypeStruct((1, DIM), jnp.float32),
            jax.ShapeDtypeStruct((1, 1), jnp.int32),
            jax.ShapeDtypeStruct((1, 1), jnp.float32),
        ],
    )(meta, rows, noisy.reshape(1, DIM))
    return clean[0], idx[0, 0], sim[0, 0]
